# Initial kernel scaffold; baseline (speedup 1.0000x reference)
import jax
import jax.numpy as jnp
from jax.experimental import pallas as pl


def kernel(x, edge_index, W1_0, b1_0, W2_0, b2_0, g_0, be_0, W1_1, b1_1, W2_1, b2_1, g_1, be_1, W1_2, b1_2, W2_2, b2_2, g_2, be_2):
    raise NotImplementedError("write your pallas kernel here")


# speedup vs baseline: 10.0358x; 10.0358x over previous
import functools

import jax
import jax.numpy as jnp
from jax import lax
from jax.experimental import pallas as pl
from jax.experimental.pallas import tpu as pltpu
from jax.experimental.pallas import tpu_sc as plsc

_N, _E, _D = 10000, 320000, 128
_NC, _NS = 2, 16
_HD = _D // _NC
_EPT = _E // _NS
_C = 125
_CH = _EPT // _C
_NB = 4
_RPT = 624
_RC = 104
_RCH = _RPT // _RC
_TAIL0 = _NS * _RPT
_TAILN = _N - _TAIL0


def _sc_aggregate(x_split, src_t, dst_t):
  mesh = plsc.VectorSubcoreMesh(core_axis_name="c", subcore_axis_name="s")

  @functools.partial(
      pl.kernel,
      out_type=jax.ShapeDtypeStruct((_NC, _N, _HD), jnp.float32),
      mesh=mesh,
      compiler_params=pltpu.CompilerParams(use_tc_tiling_on_sc=False),
      scratch_types=[
          pltpu.VMEM((_CH, _C), jnp.int32),
          pltpu.VMEM((_CH, _C), jnp.int32),
          pltpu.VMEM((_NB, _C, _HD), jnp.float32),
          pltpu.VMEM_SHARED((_N, _HD), jnp.float32),
          pltpu.SemaphoreType.DMA((_NB,)),
      ],
  )
  def agg_kernel(x_hbm, src_hbm, dst_hbm, out_hbm, src_v, dst_v, rows_v,
                 agg_sh, gsem):
    c = lax.axis_index("c")
    s = lax.axis_index("s")
    xc = x_hbm.at[c]

    pltpu.sync_copy(src_hbm.at[s], src_v)
    pltpu.sync_copy(dst_hbm.at[s], dst_v)

    r0 = s * _RPT
    for r in range(_RCH):
      buf = rows_v.at[r % _NB].at[pl.ds(0, _RC)]
      pltpu.sync_copy(xc.at[pl.ds(r0 + r * _RC, _RC)], buf)
      pltpu.sync_copy(buf, agg_sh.at[pl.ds(r0 + r * _RC, _RC)])

    @pl.when(s == _NS - 1)
    def _():
      buf = rows_v.at[0].at[pl.ds(0, _TAILN)]
      pltpu.sync_copy(xc.at[pl.ds(_TAIL0, _TAILN)], buf)
      pltpu.sync_copy(buf, agg_sh.at[pl.ds(_TAIL0, _TAILN)])

    plsc.subcore_barrier()

    for b in range(_NB):
      pltpu.async_copy(xc.at[src_v.at[b]], rows_v.at[b], gsem.at[b])

    @pl.loop(0, _CH, step=_NB)
    def _(j0):
      for b in range(_NB):
        j = j0 + b
        pltpu.make_async_copy(xc.at[src_v.at[b]], rows_v.at[b],
                              gsem.at[b]).wait()
        pltpu.sync_copy(rows_v.at[b], agg_sh.at[dst_v.at[j]], add=True)
        nj = j + _NB

        @pl.when(nj < _CH)
        def _():
          pltpu.async_copy(xc.at[src_v.at[nj]], rows_v.at[b], gsem.at[b])

    plsc.subcore_barrier()

    for r in range(_RCH):
      buf = rows_v.at[r % _NB].at[pl.ds(0, _RC)]
      pltpu.sync_copy(agg_sh.at[pl.ds(r0 + r * _RC, _RC)], buf)
      pltpu.sync_copy(buf, out_hbm.at[c, pl.ds(r0 + r * _RC, _RC)])

    @pl.when(s == _NS - 1)
    def _():
      buf = rows_v.at[0].at[pl.ds(0, _TAILN)]
      pltpu.sync_copy(agg_sh.at[pl.ds(_TAIL0, _TAILN)], buf)
      pltpu.sync_copy(buf, out_hbm.at[c, pl.ds(_TAIL0, _TAILN)])

  return agg_kernel(x_split, src_t, dst_t)


def _mlp_bn(aggL, aggH, W1, b1, W2, b2, g, be):
  def body(aL, aH, W1r, b1r, W2r, b2r, gr, ber, out):
    h = jnp.concatenate([aL[...], aH[...]], axis=1)
    z = jnp.maximum(
        jnp.dot(h, W1r[...]) + b1r[...], 0.0)
    z = jnp.dot(z, W2r[...]) + b2r[...]
    mu = jnp.mean(z, axis=0, keepdims=True)
    var = jnp.mean((z - mu) * (z - mu), axis=0, keepdims=True)
    zn = (z - mu) * lax.rsqrt(var + 1e-5) * gr[...] + ber[...]
    out[...] = jnp.maximum(zn, 0.0)

  return pl.pallas_call(
      body,
      out_shape=jax.ShapeDtypeStruct((_N, _D), jnp.float32),
  )(aggL, aggH, W1, b1.reshape(1, _D), W2, b2.reshape(1, _D),
    g.reshape(1, _D), be.reshape(1, _D))


def kernel(x, edge_index, W1_0, b1_0, W2_0, b2_0, g_0, be_0, W1_1, b1_1,
           W2_1, b2_1, g_1, be_1, W1_2, b1_2, W2_2, b2_2, g_2, be_2):
  x = x.astype(jnp.bfloat16).astype(jnp.float32)
  src_t = edge_index[0].reshape(_NS, _CH, _C)
  dst_t = edge_index[1].reshape(_NS, _CH, _C)
  params = [(W1_0, b1_0, W2_0, b2_0, g_0, be_0),
            (W1_1, b1_1, W2_1, b2_1, g_1, be_1),
            (W1_2, b1_2, W2_2, b2_2, g_2, be_2)]
  for (W1, b1, W2, b2, g, be) in params:
    x_split = jnp.stack([x[:, :_HD], x[:, _HD:]])
    agg = _sc_aggregate(x_split, src_t, dst_t)
    x = _mlp_bn(agg[0], agg[1], W1, b1, W2, b2, g, be)
  return x
